# baseline (speedup 1.0000x reference)
import functools

import jax
import jax.numpy as jnp
from jax import lax
from jax.experimental import pallas as pl
from jax.experimental.pallas import tpu as pltpu
from jax.experimental.pallas import tpu_sc as plsc

S = 512
H = 256
D = 128
MAXG = 2

_F32 = jnp.float32


def _sc_gather(token_table, gaz_table, tok_idx, fw_idx, bw_idx):
    mesh = plsc.VectorSubcoreMesh(core_axis_name="c", subcore_axis_name="s")

    @functools.partial(
        pl.kernel,
        mesh=mesh,
        out_type=[
            jax.ShapeDtypeStruct((S, D), _F32),
            jax.ShapeDtypeStruct((S * MAXG, D), _F32),
            jax.ShapeDtypeStruct((S * 6, D), _F32),
        ],
        scratch_types=[
            pltpu.VMEM((16,), jnp.int32),
            pltpu.VMEM((16, D), _F32),
            pltpu.VMEM((32,), jnp.int32),
            pltpu.VMEM((32, D), _F32),
            pltpu.VMEM((96,), jnp.int32),
            pltpu.VMEM((96, D), _F32),
            pltpu.SemaphoreType.DMA,
        ],
    )
    def gk(tok_tab, gaz_tab, t_idx, f_idx, b_idx, emb_o, fwg_o, bwg_o,
           ti_v, tr_v, fi_v, fr_v, bi_v, br_v, sem):
        wid = lax.axis_index("s") * 2 + lax.axis_index("c")
        pltpu.sync_copy(t_idx.at[pl.ds(wid * 16, 16)], ti_v)
        pltpu.async_copy(tok_tab.at[ti_v], tr_v, sem).wait()
        pltpu.sync_copy(tr_v, emb_o.at[pl.ds(wid * 16, 16)])
        pltpu.sync_copy(f_idx.at[pl.ds(wid * 32, 32)], fi_v)
        pltpu.async_copy(gaz_tab.at[fi_v], fr_v, sem).wait()
        pltpu.sync_copy(fr_v, fwg_o.at[pl.ds(wid * 32, 32)])
        pltpu.sync_copy(b_idx.at[pl.ds(wid * 96, 96)], bi_v)
        pltpu.async_copy(gaz_tab.at[bi_v], br_v, sem).wait()
        pltpu.sync_copy(br_v, bwg_o.at[pl.ds(wid * 96, 96)])

    return gk(token_table, gaz_table, tok_idx, fw_idx, bw_idx)


def _dot(a, b):
    return jnp.dot(a, b, preferred_element_type=_F32)


def _dotb(a, b):
    return jnp.dot(a.astype(jnp.bfloat16), b, preferred_element_type=_F32)


def _step(t, h, c, B0, B1, B2, M0, M1, M2,
          xw_ref, xl_ref, ge_ref, val_ref, Wh, Wwx, Wwh, bw_b, Wlc, W):
    xw = xw_ref[pl.ds(t, 1), :]
    g4 = xw + _dotb(h, Wh[:, :])
    i_g = jax.nn.sigmoid(g4[:, :H])
    f_g = jax.nn.sigmoid(g4[:, H:2 * H])
    o_g = jax.nn.sigmoid(g4[:, 2 * H:3 * H])
    g_g = jnp.tanh(g4[:, 3 * H:])

    aw = xl_ref[pl.ds(t, 1), :] + _dotb(B0, Wlc[:, :])
    ew = M0 * jnp.exp(jax.nn.sigmoid(aw))
    e0 = jnp.exp(i_g)
    s_e = jnp.sum(ew, axis=0, keepdims=True)
    s_ec = jnp.sum(ew * B0, axis=0, keepdims=True)
    anym = jnp.max(M0, axis=0, keepdims=True)
    c_multi = (e0 * g_g + s_ec) / (e0 + s_e)
    c_plain = f_g * c + i_g * g_g
    c_new = jnp.where(anym > 0.5, c_multi, c_plain)
    h_new = o_g * jnp.tanh(c_new)

    ge = ge_ref[pl.ds(W * t, W), :]
    wg = _dotb(ge, Wwx[:, :]) + _dotb(h_new, Wwh[:, :]) + bw_b[:, :]
    iw = jax.nn.sigmoid(wg[:, :H])
    fw_ = jax.nn.sigmoid(wg[:, H:2 * H])
    gw = jnp.tanh(wg[:, 2 * H:])
    cw = fw_ * c_new + iw * gw
    cw6 = cw if W == 6 else jnp.concatenate([cw, cw, cw], axis=0)

    vrow = val_ref[pl.ds(t, 1), :]
    ri = lax.broadcasted_iota(jnp.int32, (6, 6), 0)
    ci = lax.broadcasted_iota(jnp.int32, (6, 6), 1)
    eye6 = (ri == ci).astype(_F32)
    vdiag = jnp.broadcast_to(vrow, (6, 6)) * eye6
    valb = _dot(vdiag, jnp.ones((6, H), _F32))

    z4 = jnp.zeros((4, H), _F32)
    B0n = jnp.concatenate([B1[0:4], cw6[0:2]], axis=0)
    B1n = jnp.concatenate([B2[0:2], cw6[2:4], B2[4:6]], axis=0)
    B2n = jnp.concatenate([cw6[4:6], z4], axis=0)
    M0n = jnp.concatenate([M1[0:4], valb[0:2]], axis=0)
    M1n = jnp.concatenate([M2[0:2], valb[2:4], M2[4:6]], axis=0)
    M2n = jnp.concatenate([valb[4:6], z4], axis=0)
    return h_new, c_new, B0n, B1n, B2n, M0n, M1n, M2n


def _tc_body(emb, gefw, gebw, valfw, valbw,
             fwWh, fwWwx, fwWwh, fwbwb, fwWlc,
             bwWh, bwWwx, bwWwh, bwbwb, bwWlc,
             fwWx, fwb, fwWlx, fwbl, bwWx, bwb, bwWlx, bwbl,
             hs_ref, xwf, xlf, xwb, xlb):
    for ci in range(8):
        r0 = ci * 64
        e = emb[r0:r0 + 64, :]
        xwf[r0:r0 + 64, :] = _dot(e, fwWx[:, :]) + fwb[:, :]
        xlf[r0:r0 + 64, :] = _dot(e, fwWlx[:, :]) + fwbl[:, :]
        xwb[r0:r0 + 64, :] = _dot(e, bwWx[:, :]) + bwb[:, :]
        xlb[r0:r0 + 64, :] = _dot(e, bwWlx[:, :]) + bwbl[:, :]

    z1 = jnp.zeros((1, H), _F32)
    z6 = jnp.zeros((6, H), _F32)
    init = (z1, z1, z6, z6, z6, z6, z6, z6,
            z1, z1, z6, z6, z6, z6, z6, z6)

    def body(t, carry):
        (hf, cf, B0f, B1f, B2f, M0f, M1f, M2f,
         hb, cb, B0b, B1b, B2b, M0b, M1b, M2b) = carry
        nf = _step(t, hf, cf, B0f, B1f, B2f, M0f, M1f, M2f,
                   xwf, xlf, gefw, valfw, fwWh, fwWwx, fwWwh, fwbwb, fwWlc, 2)
        p = S - 1 - t
        nb = _step(p, hb, cb, B0b, B1b, B2b, M0b, M1b, M2b,
                   xwb, xlb, gebw, valbw, bwWh, bwWwx, bwWwh, bwbwb, bwWlc, 6)
        hs_ref[pl.ds(t, 1), 0:H] = nf[0]
        hs_ref[pl.ds(p, 1), H:2 * H] = nb[0]
        return (*nf, *nb)

    lax.fori_loop(0, S, body, init)


def _tc_lattice(emb, gefw, gebw, valfw, valbw,
                fwWh, fwWwx, fwWwh, fwbwb, fwWlc,
                bwWh, bwWwx, bwWwh, bwbwb, bwWlc,
                fwWx, fwb, fwWlx, fwbl, bwWx, bwb, bwWlx, bwbl):
    return pl.pallas_call(
        _tc_body,
        out_shape=jax.ShapeDtypeStruct((S, 2 * H), _F32),
        scratch_shapes=[
            pltpu.VMEM((S, 4 * H), _F32),
            pltpu.VMEM((S, H), _F32),
            pltpu.VMEM((S, 4 * H), _F32),
            pltpu.VMEM((S, H), _F32),
        ],
    )(emb, gefw, gebw, valfw, valbw,
      fwWh, fwWwx, fwWwh, fwbwb, fwWlc,
      bwWh, bwWwx, bwWwh, bwbwb, bwWlc,
      fwWx, fwb, fwWlx, fwbl, bwWx, bwb, bwWlx, bwbl)


def kernel(tokens, gaz_ids, gaz_lengths, token_table, gaz_table,
           fw_Wx, fw_Wh, fw_b, fw_Wwx, fw_Wwh, fw_bw, fw_Wlx, fw_Wlc, fw_bl,
           bw_Wx, bw_Wh, bw_b, bw_Wwx, bw_Wwh, bw_bw, bw_Wlx, bw_Wlc, bw_bl):
    tok_idx = tokens.reshape(S).astype(jnp.int32)
    gi = gaz_ids.astype(jnp.int32)
    gl = gaz_lengths.astype(jnp.int32)
    pos = jnp.arange(S, dtype=jnp.int32)[:, None]

    fw_idx = gi.reshape(S * MAXG)

    bw_cols, vf_cols, vb_cols = [], [], []
    for dd in (1, 2, 3):
        gi_s = jnp.concatenate([jnp.zeros((dd, MAXG), jnp.int32), gi[:S - dd]], axis=0)
        gl_s = jnp.concatenate([jnp.zeros((dd, MAXG), jnp.int32), gl[:S - dd]], axis=0)
        bw_cols.append(gi_s)
        vf_cols.append((gl == dd) & (pos + dd < S))
        vb_cols.append((pos >= dd) & (gl_s == dd))
    bw_idx = jnp.concatenate(bw_cols, axis=1).reshape(S * 6)
    valfw = jnp.concatenate(vf_cols, axis=1).astype(_F32)
    valbw = jnp.concatenate(vb_cols, axis=1).astype(_F32)

    emb, gefw, gebw = _sc_gather(token_table, gaz_table, tok_idx, fw_idx, bw_idx)

    bf = jnp.bfloat16
    hs = _tc_lattice(
        emb, gefw, gebw, valfw, valbw,
        fw_Wh.astype(bf), fw_Wwx.astype(bf), fw_Wwh.astype(bf),
        fw_bw.reshape(1, 3 * H), fw_Wlc.astype(bf),
        bw_Wh.astype(bf), bw_Wwx.astype(bf), bw_Wwh.astype(bf),
        bw_bw.reshape(1, 3 * H), bw_Wlc.astype(bf),
        fw_Wx, fw_b.reshape(1, 4 * H), fw_Wlx, fw_bl.reshape(1, H),
        bw_Wx, bw_b.reshape(1, 4 * H), bw_Wlx, bw_bl.reshape(1, H))
    return hs[None, :, :]


# speedup vs baseline: 1.2743x; 1.2743x over previous
import functools

import jax
import jax.numpy as jnp
from jax import lax
from jax.experimental import pallas as pl
from jax.experimental.pallas import tpu as pltpu
from jax.experimental.pallas import tpu_sc as plsc

S = 512
H = 256
D = 128
MAXG = 2

_F32 = jnp.float32


def _sc_gather(token_table, gaz_table, tok_idx, fw_idx, bw_idx):
    mesh = plsc.VectorSubcoreMesh(core_axis_name="c", subcore_axis_name="s")

    @functools.partial(
        pl.kernel,
        mesh=mesh,
        out_type=[
            jax.ShapeDtypeStruct((S, D), _F32),
            jax.ShapeDtypeStruct((S * MAXG, D), _F32),
            jax.ShapeDtypeStruct((S * 6, D), _F32),
        ],
        scratch_types=[
            pltpu.VMEM((16,), jnp.int32),
            pltpu.VMEM((16, D), _F32),
            pltpu.VMEM((32,), jnp.int32),
            pltpu.VMEM((32, D), _F32),
            pltpu.VMEM((96,), jnp.int32),
            pltpu.VMEM((96, D), _F32),
            pltpu.SemaphoreType.DMA,
        ],
    )
    def gk(tok_tab, gaz_tab, t_idx, f_idx, b_idx, emb_o, fwg_o, bwg_o,
           ti_v, tr_v, fi_v, fr_v, bi_v, br_v, sem):
        wid = lax.axis_index("s") * 2 + lax.axis_index("c")
        pltpu.sync_copy(t_idx.at[pl.ds(wid * 16, 16)], ti_v)
        pltpu.async_copy(tok_tab.at[ti_v], tr_v, sem).wait()
        pltpu.sync_copy(tr_v, emb_o.at[pl.ds(wid * 16, 16)])
        pltpu.sync_copy(f_idx.at[pl.ds(wid * 32, 32)], fi_v)
        pltpu.async_copy(gaz_tab.at[fi_v], fr_v, sem).wait()
        pltpu.sync_copy(fr_v, fwg_o.at[pl.ds(wid * 32, 32)])
        pltpu.sync_copy(b_idx.at[pl.ds(wid * 96, 96)], bi_v)
        pltpu.async_copy(gaz_tab.at[bi_v], br_v, sem).wait()
        pltpu.sync_copy(br_v, bwg_o.at[pl.ds(wid * 96, 96)])

    return gk(token_table, gaz_table, tok_idx, fw_idx, bw_idx)


def _dot(a, b):
    return jnp.dot(a, b, preferred_element_type=_F32)


def _dotb(a, b):
    return jnp.dot(a.astype(jnp.bfloat16), b, preferred_element_type=_F32)


def _valb(val_ref, t):
    vrow = val_ref[pl.ds(t, 1), :]
    ri = lax.broadcasted_iota(jnp.int32, (6, 6), 0)
    ci = lax.broadcasted_iota(jnp.int32, (6, 6), 1)
    eye6 = (ri == ci).astype(_F32)
    vdiag = jnp.broadcast_to(vrow, (6, 6)) * eye6
    return _dot(vdiag, jnp.ones((6, H), _F32))


def _gates(g4):
    sg = jax.nn.sigmoid(g4[:, :3 * H])
    return sg[:, :H], sg[:, H:2 * H], sg[:, 2 * H:], jnp.tanh(g4[:, 3 * H:])


def _cnew(c, B0, M0, aw, i_g, f_g, g_g):
    ew = M0 * jnp.exp(jax.nn.sigmoid(aw))
    e0 = jnp.exp(i_g)
    s_e = jnp.sum(ew, axis=0, keepdims=True)
    s_ec = jnp.sum(ew * B0, axis=0, keepdims=True)
    anym = jnp.max(M0, axis=0, keepdims=True)
    c_multi = (e0 * g_g + s_ec) / (e0 + s_e)
    c_plain = f_g * c + i_g * g_g
    return jnp.where(anym > 0.5, c_multi, c_plain)


def _wordcells(wg, c_new):
    sg = jax.nn.sigmoid(wg[:, :2 * H])
    iw, fw_, gw = sg[:, :H], sg[:, H:], jnp.tanh(wg[:, 2 * H:])
    return fw_ * c_new + iw * gw


def _bupdate(B1, B2, M1, M2, cw6, valb):
    z4 = jnp.zeros((4, H), _F32)
    B0n = jnp.concatenate([B1[0:4], cw6[0:2]], axis=0)
    B1n = jnp.concatenate([B2[0:2], cw6[2:4], B2[4:6]], axis=0)
    B2n = jnp.concatenate([cw6[4:6], z4], axis=0)
    M0n = jnp.concatenate([M1[0:4], valb[0:2]], axis=0)
    M1n = jnp.concatenate([M2[0:2], valb[2:4], M2[4:6]], axis=0)
    M2n = jnp.concatenate([valb[4:6], z4], axis=0)
    return B0n, B1n, B2n, M0n, M1n, M2n


def _tc_body(emb, gefw, gebw, valfw, valbw,
             fwWh, fwWwx, fwWwh, fwbwb, fwWlc,
             bwWh, bwWwx, bwWwh, bwbwb, bwWlc,
             fwWx, fwb, fwWlx, fwbl, bwWx, bwb, bwWlx, bwbl,
             hs_ref, xwf, xlf, xwb, xlb):
    for ci in range(8):
        r0 = ci * 64
        e = emb[r0:r0 + 64, :]
        xwf[r0:r0 + 64, :] = _dot(e, fwWx[:, :]) + fwb[:, :]
        xlf[r0:r0 + 64, :] = _dot(e, fwWlx[:, :]) + fwbl[:, :]
        xwb[r0:r0 + 64, :] = _dot(e, bwWx[:, :]) + bwb[:, :]
        xlb[r0:r0 + 64, :] = _dot(e, bwWlx[:, :]) + bwbl[:, :]

    z1 = jnp.zeros((1, H), _F32)
    z6 = jnp.zeros((6, H), _F32)
    init = (z1, z1, z6, z6, z6, z6, z6, z6,
            z1, z1, z6, z6, z6, z6, z6, z6)

    def body(t, carry):
        (hf, cf, B0f, B1f, B2f, M0f, M1f, M2f,
         hb, cb, B0b, B1b, B2b, M0b, M1b, M2b) = carry
        p = S - 1 - t
        xw_f = xwf[pl.ds(t, 1), :]
        xw_b = xwb[pl.ds(p, 1), :]
        xl_f = xlf[pl.ds(t, 1), :]
        xl_b = xlb[pl.ds(p, 1), :]
        gx_f = _dotb(gefw[pl.ds(2 * t, 2), :], fwWwx[:, :]) + fwbwb[:, :]
        gx_b = _dotb(gebw[pl.ds(6 * p, 6), :], bwWwx[:, :]) + bwbwb[:, :]
        vb_f = _valb(valfw, t)
        vb_b = _valb(valbw, p)
        g4f = xw_f + _dotb(hf, fwWh[:, :])
        g4b = xw_b + _dotb(hb, bwWh[:, :])
        awf = xl_f + _dotb(B0f, fwWlc[:, :])
        awb = xl_b + _dotb(B0b, bwWlc[:, :])
        if_, ff, of, gf = _gates(g4f)
        ib_, fb, ob, gb = _gates(g4b)
        cf_n = _cnew(cf, B0f, M0f, awf, if_, ff, gf)
        cb_n = _cnew(cb, B0b, M0b, awb, ib_, fb, gb)
        hf_n = of * jnp.tanh(cf_n)
        hb_n = ob * jnp.tanh(cb_n)
        wgf = gx_f + _dotb(hf_n, fwWwh[:, :])
        wgb = gx_b + _dotb(hb_n, bwWwh[:, :])
        cwf = _wordcells(wgf, cf_n)
        cwb = _wordcells(wgb, cb_n)
        cw6f = jnp.concatenate([cwf, cwf, cwf], axis=0)
        nf2 = _bupdate(B1f, B2f, M1f, M2f, cw6f, vb_f)
        nb2 = _bupdate(B1b, B2b, M1b, M2b, cwb, vb_b)
        hs_ref[pl.ds(t, 1), 0:H] = hf_n
        hs_ref[pl.ds(p, 1), H:2 * H] = hb_n
        return (hf_n, cf_n, *nf2, hb_n, cb_n, *nb2)

    lax.fori_loop(0, S, body, init)


def _tc_lattice(emb, gefw, gebw, valfw, valbw,
                fwWh, fwWwx, fwWwh, fwbwb, fwWlc,
                bwWh, bwWwx, bwWwh, bwbwb, bwWlc,
                fwWx, fwb, fwWlx, fwbl, bwWx, bwb, bwWlx, bwbl):
    return pl.pallas_call(
        _tc_body,
        out_shape=jax.ShapeDtypeStruct((S, 2 * H), _F32),
        scratch_shapes=[
            pltpu.VMEM((S, 4 * H), _F32),
            pltpu.VMEM((S, H), _F32),
            pltpu.VMEM((S, 4 * H), _F32),
            pltpu.VMEM((S, H), _F32),
        ],
    )(emb, gefw, gebw, valfw, valbw,
      fwWh, fwWwx, fwWwh, fwbwb, fwWlc,
      bwWh, bwWwx, bwWwh, bwbwb, bwWlc,
      fwWx, fwb, fwWlx, fwbl, bwWx, bwb, bwWlx, bwbl)


def kernel(tokens, gaz_ids, gaz_lengths, token_table, gaz_table,
           fw_Wx, fw_Wh, fw_b, fw_Wwx, fw_Wwh, fw_bw, fw_Wlx, fw_Wlc, fw_bl,
           bw_Wx, bw_Wh, bw_b, bw_Wwx, bw_Wwh, bw_bw, bw_Wlx, bw_Wlc, bw_bl):
    tok_idx = tokens.reshape(S).astype(jnp.int32)
    gi = gaz_ids.astype(jnp.int32)
    gl = gaz_lengths.astype(jnp.int32)
    pos = jnp.arange(S, dtype=jnp.int32)[:, None]

    fw_idx = gi.reshape(S * MAXG)

    bw_cols, vf_cols, vb_cols = [], [], []
    for dd in (1, 2, 3):
        gi_s = jnp.concatenate([jnp.zeros((dd, MAXG), jnp.int32), gi[:S - dd]], axis=0)
        gl_s = jnp.concatenate([jnp.zeros((dd, MAXG), jnp.int32), gl[:S - dd]], axis=0)
        bw_cols.append(gi_s)
        vf_cols.append((gl == dd) & (pos + dd < S))
        vb_cols.append((pos >= dd) & (gl_s == dd))
    bw_idx = jnp.concatenate(bw_cols, axis=1).reshape(S * 6)
    valfw = jnp.concatenate(vf_cols, axis=1).astype(_F32)
    valbw = jnp.concatenate(vb_cols, axis=1).astype(_F32)

    emb, gefw, gebw = _sc_gather(token_table, gaz_table, tok_idx, fw_idx, bw_idx)

    bf = jnp.bfloat16
    hs = _tc_lattice(
        emb, gefw, gebw, valfw, valbw,
        fw_Wh.astype(bf), fw_Wwx.astype(bf), fw_Wwh.astype(bf),
        fw_bw.reshape(1, 3 * H), fw_Wlc.astype(bf),
        bw_Wh.astype(bf), bw_Wwx.astype(bf), bw_Wwh.astype(bf),
        bw_bw.reshape(1, 3 * H), bw_Wlc.astype(bf),
        fw_Wx, fw_b.reshape(1, 4 * H), fw_Wlx, fw_bl.reshape(1, H),
        bw_Wx, bw_b.reshape(1, 4 * H), bw_Wlx, bw_bl.reshape(1, H))
    return hs[None, :, :]


# speedup vs baseline: 1.5426x; 1.2105x over previous
import functools

import jax
import jax.numpy as jnp
from jax import lax
from jax.experimental import pallas as pl
from jax.experimental.pallas import tpu as pltpu
from jax.experimental.pallas import tpu_sc as plsc

S = 512
H = 256
D = 128
MAXG = 2

_F32 = jnp.float32


def _sc_gather(token_table, gaz_table, tok_idx, fw_idx, bw_idx):
    mesh = plsc.VectorSubcoreMesh(core_axis_name="c", subcore_axis_name="s")

    @functools.partial(
        pl.kernel,
        mesh=mesh,
        out_type=[
            jax.ShapeDtypeStruct((S, D), _F32),
            jax.ShapeDtypeStruct((S * MAXG, D), _F32),
            jax.ShapeDtypeStruct((S * 6, D), _F32),
        ],
        scratch_types=[
            pltpu.VMEM((16,), jnp.int32),
            pltpu.VMEM((16, D), _F32),
            pltpu.VMEM((32,), jnp.int32),
            pltpu.VMEM((32, D), _F32),
            pltpu.VMEM((96,), jnp.int32),
            pltpu.VMEM((96, D), _F32),
            pltpu.SemaphoreType.DMA,
        ],
    )
    def gk(tok_tab, gaz_tab, t_idx, f_idx, b_idx, emb_o, fwg_o, bwg_o,
           ti_v, tr_v, fi_v, fr_v, bi_v, br_v, sem):
        wid = lax.axis_index("s") * 2 + lax.axis_index("c")
        pltpu.sync_copy(t_idx.at[pl.ds(wid * 16, 16)], ti_v)
        pltpu.async_copy(tok_tab.at[ti_v], tr_v, sem).wait()
        pltpu.sync_copy(tr_v, emb_o.at[pl.ds(wid * 16, 16)])
        pltpu.sync_copy(f_idx.at[pl.ds(wid * 32, 32)], fi_v)
        pltpu.async_copy(gaz_tab.at[fi_v], fr_v, sem).wait()
        pltpu.sync_copy(fr_v, fwg_o.at[pl.ds(wid * 32, 32)])
        pltpu.sync_copy(b_idx.at[pl.ds(wid * 96, 96)], bi_v)
        pltpu.async_copy(gaz_tab.at[bi_v], br_v, sem).wait()
        pltpu.sync_copy(br_v, bwg_o.at[pl.ds(wid * 96, 96)])

    return gk(token_table, gaz_table, tok_idx, fw_idx, bw_idx)


def _dot(a, b):
    return jnp.dot(a, b, preferred_element_type=_F32)


def _dotb(a, b):
    return jnp.dot(a.astype(jnp.bfloat16), b, preferred_element_type=_F32)


def _valb(val_ref, t):
    vrow = val_ref[pl.ds(t, 1), :]
    ri = lax.broadcasted_iota(jnp.int32, (6, 6), 0)
    ci = lax.broadcasted_iota(jnp.int32, (6, 6), 1)
    eye6 = (ri == ci).astype(_F32)
    vdiag = jnp.broadcast_to(vrow, (6, 6)) * eye6
    return _dot(vdiag, jnp.ones((6, H), _F32))


def _gates(g4):
    sg = jax.nn.sigmoid(g4[:, :3 * H])
    return sg[:, :H], sg[:, H:2 * H], sg[:, 2 * H:], jnp.tanh(g4[:, 3 * H:])


def _cnew(c, B0, M0, aw, i_g, f_g, g_g):
    ew = M0 * jnp.exp(jax.nn.sigmoid(aw))
    e0 = jnp.exp(i_g)
    s_e = jnp.sum(ew, axis=0, keepdims=True)
    s_ec = jnp.sum(ew * B0, axis=0, keepdims=True)
    anym = jnp.max(M0, axis=0, keepdims=True)
    c_multi = (e0 * g_g + s_ec) / (e0 + s_e)
    c_plain = f_g * c + i_g * g_g
    return jnp.where(anym > 0.5, c_multi, c_plain)


def _wordcells(wg, c_new):
    sg = jax.nn.sigmoid(wg[:, :2 * H])
    iw, fw_, gw = sg[:, :H], sg[:, H:], jnp.tanh(wg[:, 2 * H:])
    return fw_ * c_new + iw * gw


def _bupdate(B1, B2, M1, M2, cw6, valb):
    z4 = jnp.zeros((4, H), _F32)
    B0n = jnp.concatenate([B1[0:4], cw6[0:2]], axis=0)
    B1n = jnp.concatenate([B2[0:2], cw6[2:4], B2[4:6]], axis=0)
    B2n = jnp.concatenate([cw6[4:6], z4], axis=0)
    M0n = jnp.concatenate([M1[0:4], valb[0:2]], axis=0)
    M1n = jnp.concatenate([M2[0:2], valb[2:4], M2[4:6]], axis=0)
    M2n = jnp.concatenate([valb[4:6], z4], axis=0)
    return B0n, B1n, B2n, M0n, M1n, M2n


def _tc_body(emb, gefw, gebw, valfw, valbw,
             fwWh, fwWwx, fwWwh, fwbwb, fwWlc,
             bwWh, bwWwx, bwWwh, bwbwb, bwWlc,
             fwWx, fwb, fwWlx, fwbl, bwWx, bwb, bwWlx, bwbl,
             hs_ref, xwf, xlf, xwb, xlb):
    for ci in range(8):
        r0 = ci * 64
        e = emb[r0:r0 + 64, :]
        xwf[r0:r0 + 64, :] = _dot(e, fwWx[:, :]) + fwb[:, :]
        xlf[r0:r0 + 64, :] = _dot(e, fwWlx[:, :]) + fwbl[:, :]
        xwb[r0:r0 + 64, :] = _dot(e, bwWx[:, :]) + bwb[:, :]
        xlb[r0:r0 + 64, :] = _dot(e, bwWlx[:, :]) + bwbl[:, :]

    z1 = jnp.zeros((1, H), _F32)
    z6 = jnp.zeros((6, H), _F32)
    init = (z1, z1, z6, z6, z6, z6, z6, z6,
            xwf[0:1, :], jnp.broadcast_to(xlf[0:1, :], (6, H)),
            z1, z1, z6, z6, z6, z6, z6, z6,
            xwb[S - 1:S, :], jnp.broadcast_to(xlb[S - 1:S, :], (6, H)))

    def body(t, carry):
        (hf, cf, B0f, B1f, B2f, M0f, M1f, M2f, g4f, awf,
         hb, cb, B0b, B1b, B2b, M0b, M1b, M2b, g4b, awb) = carry
        p = S - 1 - t
        tn = jnp.minimum(t + 1, S - 1)
        pn = jnp.maximum(p - 1, 0)
        if_, ff, of, gf = _gates(g4f)
        ib_, fb, ob, gb = _gates(g4b)
        cf_n = _cnew(cf, B0f, M0f, awf, if_, ff, gf)
        cb_n = _cnew(cb, B0b, M0b, awb, ib_, fb, gb)
        hf_n = of * jnp.tanh(cf_n)
        hb_n = ob * jnp.tanh(cb_n)
        hs_ref[pl.ds(t, 1), 0:H] = hf_n
        hs_ref[pl.ds(p, 1), H:2 * H] = hb_n
        gx_f = _dotb(gefw[pl.ds(2 * t, 2), :], fwWwx[:, :]) + fwbwb[:, :]
        gx_b = _dotb(gebw[pl.ds(6 * p, 6), :], bwWwx[:, :]) + bwbwb[:, :]
        vb_f = _valb(valfw, t)
        vb_b = _valb(valbw, p)
        wgf = gx_f + _dotb(hf_n, fwWwh[:, :])
        wgb = gx_b + _dotb(hb_n, bwWwh[:, :])
        cwf = _wordcells(wgf, cf_n)
        cwb = _wordcells(wgb, cb_n)
        cw6f = jnp.concatenate([cwf, cwf, cwf], axis=0)
        nf2 = _bupdate(B1f, B2f, M1f, M2f, cw6f, vb_f)
        nb2 = _bupdate(B1b, B2b, M1b, M2b, cwb, vb_b)
        g4f_n = xwf[pl.ds(tn, 1), :] + _dotb(hf_n, fwWh[:, :])
        g4b_n = xwb[pl.ds(pn, 1), :] + _dotb(hb_n, bwWh[:, :])
        awf_n = xlf[pl.ds(tn, 1), :] + jnp.concatenate(
            [_dotb(B1f[0:4], fwWlc[:, :]), _dotb(cwf[0:2], fwWlc[:, :])], axis=0)
        awb_n = xlb[pl.ds(pn, 1), :] + jnp.concatenate(
            [_dotb(B1b[0:4], bwWlc[:, :]), _dotb(cwb[0:2], bwWlc[:, :])], axis=0)
        return (hf_n, cf_n, *nf2, g4f_n, awf_n,
                hb_n, cb_n, *nb2, g4b_n, awb_n)

    lax.fori_loop(0, S, body, init)


def _tc_lattice(emb, gefw, gebw, valfw, valbw,
                fwWh, fwWwx, fwWwh, fwbwb, fwWlc,
                bwWh, bwWwx, bwWwh, bwbwb, bwWlc,
                fwWx, fwb, fwWlx, fwbl, bwWx, bwb, bwWlx, bwbl):
    return pl.pallas_call(
        _tc_body,
        out_shape=jax.ShapeDtypeStruct((S, 2 * H), _F32),
        scratch_shapes=[
            pltpu.VMEM((S, 4 * H), _F32),
            pltpu.VMEM((S, H), _F32),
            pltpu.VMEM((S, 4 * H), _F32),
            pltpu.VMEM((S, H), _F32),
        ],
    )(emb, gefw, gebw, valfw, valbw,
      fwWh, fwWwx, fwWwh, fwbwb, fwWlc,
      bwWh, bwWwx, bwWwh, bwbwb, bwWlc,
      fwWx, fwb, fwWlx, fwbl, bwWx, bwb, bwWlx, bwbl)


def kernel(tokens, gaz_ids, gaz_lengths, token_table, gaz_table,
           fw_Wx, fw_Wh, fw_b, fw_Wwx, fw_Wwh, fw_bw, fw_Wlx, fw_Wlc, fw_bl,
           bw_Wx, bw_Wh, bw_b, bw_Wwx, bw_Wwh, bw_bw, bw_Wlx, bw_Wlc, bw_bl):
    tok_idx = tokens.reshape(S).astype(jnp.int32)
    gi = gaz_ids.astype(jnp.int32)
    gl = gaz_lengths.astype(jnp.int32)
    pos = jnp.arange(S, dtype=jnp.int32)[:, None]

    fw_idx = gi.reshape(S * MAXG)

    bw_cols, vf_cols, vb_cols = [], [], []
    for dd in (1, 2, 3):
        gi_s = jnp.concatenate([jnp.zeros((dd, MAXG), jnp.int32), gi[:S - dd]], axis=0)
        gl_s = jnp.concatenate([jnp.zeros((dd, MAXG), jnp.int32), gl[:S - dd]], axis=0)
        bw_cols.append(gi_s)
        vf_cols.append((gl == dd) & (pos + dd < S))
        vb_cols.append((pos >= dd) & (gl_s == dd))
    bw_idx = jnp.concatenate(bw_cols, axis=1).reshape(S * 6)
    valfw = jnp.concatenate(vf_cols, axis=1).astype(_F32)
    valbw = jnp.concatenate(vb_cols, axis=1).astype(_F32)

    emb, gefw, gebw = _sc_gather(token_table, gaz_table, tok_idx, fw_idx, bw_idx)

    bf = jnp.bfloat16
    hs = _tc_lattice(
        emb, gefw, gebw, valfw, valbw,
        fw_Wh.astype(bf), fw_Wwx.astype(bf), fw_Wwh.astype(bf),
        fw_bw.reshape(1, 3 * H), fw_Wlc.astype(bf),
        bw_Wh.astype(bf), bw_Wwx.astype(bf), bw_Wwh.astype(bf),
        bw_bw.reshape(1, 3 * H), bw_Wlc.astype(bf),
        fw_Wx, fw_b.reshape(1, 4 * H), fw_Wlx, fw_bl.reshape(1, H),
        bw_Wx, bw_b.reshape(1, 4 * H), bw_Wlx, bw_bl.reshape(1, H))
    return hs[None, :, :]
